# baseline (speedup 1.0000x reference)
import jax
import jax.numpy as jnp
from jax import lax
from jax.experimental import pallas as pl
from jax.experimental.pallas import tpu as pltpu
from jax.experimental.pallas import tpu_sc as plsc

N_USERS = 1000000
N_ITEMS = 1000000
N_FACTORS = 64
BATCH = 16384
Y_SCALE = 5.0

NC = 2
NS = 16
NW = NC * NS
B_PER_W = BATCH // NW
IDX_CHUNK = 128
N_CHUNKS = B_PER_W // IDX_CHUNK
L = 16
STRIDE = B_PER_W + 1
GROUPS = B_PER_W // L


def _body(users_hbm, items_hbm, uw_hbm, iw_hbm, ub_hbm, ib_hbm, out_hbm,
          uidx_v, iidx_v, ubrow_idx_v, ibrow_idx_v,
          urows_v, irows_v, ubrows_v, ibrows_v, stage_v, out_v, sem):
    wid = lax.axis_index("s") * NC + lax.axis_index("c")

    pltpu.sync_copy(users_hbm.at[wid], uidx_v)
    pltpu.sync_copy(items_hbm.at[wid], iidx_v)

    def shift_body(g, carry):
        c, o = g // (IDX_CHUNK // L), (g % (IDX_CHUNK // L)) * L
        ubrow_idx_v[c, pl.ds(o, L)] = lax.shift_right_logical(
            uidx_v[c, pl.ds(o, L)], 4)
        ibrow_idx_v[c, pl.ds(o, L)] = lax.shift_right_logical(
            iidx_v[c, pl.ds(o, L)], 4)
        return carry

    lax.fori_loop(0, GROUPS, shift_body, 0)

    copies = []
    for c in range(N_CHUNKS):
        sl = pl.ds(c * IDX_CHUNK, IDX_CHUNK)
        copies.append(pltpu.async_copy(uw_hbm.at[uidx_v.at[c]],
                                       urows_v.at[sl], sem))
        copies.append(pltpu.async_copy(iw_hbm.at[iidx_v.at[c]],
                                       irows_v.at[sl], sem))
        copies.append(pltpu.async_copy(ub_hbm.at[ubrow_idx_v.at[c]],
                                       ubrows_v.at[sl], sem))
        copies.append(pltpu.async_copy(ib_hbm.at[ibrow_idx_v.at[c]],
                                       ibrows_v.at[sl], sem))
    for cp in copies:
        cp.wait()

    lane = lax.iota(jnp.int32, L)
    lane_base = lane * STRIDE

    def row_body(g, carry):
        for k in range(4):
            r = g * 4 + k
            p = (urows_v[r, pl.ds(0, L)] * irows_v[r, pl.ds(0, L)]
                 + urows_v[r, pl.ds(L, L)] * irows_v[r, pl.ds(L, L)]
                 + urows_v[r, pl.ds(2 * L, L)] * irows_v[r, pl.ds(2 * L, L)]
                 + urows_v[r, pl.ds(3 * L, L)] * irows_v[r, pl.ds(3 * L, L)])
            plsc.store_scatter(stage_v, [lane_base + r], p)
        return carry

    lax.fori_loop(0, B_PER_W // 4, row_body, 0)

    def epi_body(g, carry):
        c, o = g // (IDX_CHUNK // L), (g % (IDX_CHUNK // L)) * L
        u_col = lax.bitwise_and(uidx_v[c, pl.ds(o, L)], 15)
        i_col = lax.bitwise_and(iidx_v[c, pl.ds(o, L)], 15)
        row_vec = g * L + lane
        ub = plsc.load_gather(ubrows_v, [row_vec, u_col])
        ib = plsc.load_gather(ibrows_v, [row_vec, i_col])
        acc = ub + ib
        for l in range(L):
            acc = acc + stage_v[pl.ds(l * STRIDE + g * L, L)]
        out_v[pl.ds(g * L, L)] = Y_SCALE / (1.0 + jnp.exp(-acc))
        return carry

    lax.fori_loop(0, GROUPS, epi_body, 0)

    pltpu.sync_copy(out_v, out_hbm.at[pl.ds(wid * B_PER_W, B_PER_W)])


@jax.jit
def _run(users, items, u_weight, i_weight, u_bias, i_bias):
    mesh = plsc.VectorSubcoreMesh(core_axis_name="c", subcore_axis_name="s",
                                  num_cores=NC, num_subcores=NS)
    f = pl.kernel(
        _body,
        out_type=jax.ShapeDtypeStruct((BATCH,), jnp.float32),
        mesh=mesh,
        compiler_params=pltpu.CompilerParams(needs_layout_passes=False,
                                             use_tc_tiling_on_sc=False),
        scratch_types=[
            pltpu.VMEM((N_CHUNKS, IDX_CHUNK), jnp.int32),
            pltpu.VMEM((N_CHUNKS, IDX_CHUNK), jnp.int32),
            pltpu.VMEM((N_CHUNKS, IDX_CHUNK), jnp.int32),
            pltpu.VMEM((N_CHUNKS, IDX_CHUNK), jnp.int32),
            pltpu.VMEM((B_PER_W, N_FACTORS), jnp.float32),
            pltpu.VMEM((B_PER_W, N_FACTORS), jnp.float32),
            pltpu.VMEM((B_PER_W, L), jnp.float32),
            pltpu.VMEM((B_PER_W, L), jnp.float32),
            pltpu.VMEM((L * STRIDE,), jnp.float32),
            pltpu.VMEM((B_PER_W,), jnp.float32),
            pltpu.SemaphoreType.DMA,
        ],
    )
    return f(users.reshape(NW, N_CHUNKS, IDX_CHUNK),
             items.reshape(NW, N_CHUNKS, IDX_CHUNK),
             u_weight, i_weight,
             u_bias.reshape(N_USERS // L, L), i_bias.reshape(N_ITEMS // L, L))


def kernel(users, items, u_weight, i_weight, u_bias, i_bias):
    return _run(users.astype(jnp.int32), items.astype(jnp.int32),
                u_weight, i_weight, u_bias, i_bias)
